# Initial kernel scaffold; baseline (speedup 1.0000x reference)
import jax
import jax.numpy as jnp
from jax.experimental import pallas as pl


def kernel(identity_image, landmark_sequence, w1, b1, w2, b2, w3, b3, w4, b4):
    raise NotImplementedError("write your pallas kernel here")


# speedup vs baseline: 1.0850x; 1.0850x over previous
import jax
import jax.numpy as jnp
from jax.experimental import pallas as pl
from jax.experimental.pallas import tpu as pltpu

HW = 96


def _im2col_s2(xp, oh, ow, k):
    H, W, C = xp.shape
    r = xp.reshape(H // 2, 2, W // 2, 2, C)
    planes = {(py, px): r[:, py, :, px, :]
              for py in (0, 1) for px in (0, 1)}
    cols = []
    for ky in range(k):
        for kx in range(k):
            p = planes[(ky % 2, kx % 2)]
            oy, ox = ky // 2, kx // 2
            cols.append(p[oy:oy + oh, ox:ox + ow, :].reshape(oh * ow, -1))
    return jnp.concatenate(cols, axis=1)


def _deconv_classes(xp, oh, ow):
    out = {}
    for ry in range(2):
        for rx in range(2):
            cols = []
            for dy in range(2):
                for dx in range(2):
                    cols.append(xp[ry + dy:ry + dy + oh, rx + dx:rx + dx + ow, :]
                                .reshape(oh * ow, -1))
            out[(ry, rx)] = jnp.concatenate(cols, axis=1)
    return out


def _interleave(c00, c01, c10, c11, h, w, ch):
    r0 = jnp.stack([c00, c01], axis=2)
    r1 = jnp.stack([c10, c11], axis=2)
    return jnp.stack([r0, r1], axis=1).reshape(2 * h, 2 * w, ch)


def _frame_kernel(ident_ref, lm_ref, w1_ref, b1_ref, w2_ref, b2_ref,
                  w3_ref, b3_ref, w4_ref, b4_ref, out_ref):
    f32 = jnp.float32

    lm = lm_ref[0] * (HW - 1)
    idx = lm.astype(jnp.int32)
    x, y = idx[0:1, :], idx[1:2, :]
    valid = (y >= 0) & (y < HW) & (x >= 0) & (x < HW)
    y_s = jnp.where(valid, y, -1)
    x_s = jnp.where(valid, x, -1)
    rows = jax.lax.broadcasted_iota(jnp.int32, (HW, 68), 0)
    rowm = (rows == y_s).astype(f32)
    colm = (rows == x_s).astype(f32)
    counts = jnp.dot(rowm, colm.T, preferred_element_type=f32)
    hm = (counts > 0).astype(f32)

    ident = ident_ref[0].transpose(1, 2, 0)
    x0 = jnp.concatenate([ident, hm[:, :, None]], axis=2)

    xp = jnp.pad(x0, ((1, 1), (1, 1), (0, 0)))
    x1 = _im2col_s2(xp, 48, 48, 4)
    y1 = jnp.maximum(jnp.dot(x1, w1_ref[...], preferred_element_type=f32)
                     + b1_ref[...], 0.0)
    y1 = y1.reshape(48, 48, 64)

    y1p = jnp.pad(y1, ((1, 1), (1, 1), (0, 0)))
    x2 = _im2col_s2(y1p, 24, 24, 4)
    y2 = jnp.maximum(jnp.dot(x2, w2_ref[...], preferred_element_type=f32)
                     + b2_ref[...], 0.0)
    y2 = y2.reshape(24, 24, 128)

    y2p = jnp.pad(y2, ((1, 1), (1, 1), (0, 0)))
    xc = _deconv_classes(y2p, 24, 24)
    yc = {}
    for i, (ry, rx) in enumerate([(0, 0), (0, 1), (1, 0), (1, 1)]):
        z = jnp.dot(xc[(ry, rx)], w3_ref[i], preferred_element_type=f32)
        yc[(ry, rx)] = jnp.maximum(z + b3_ref[...], 0.0).reshape(24, 24, 64)
    y3 = _interleave(yc[(0, 0)], yc[(0, 1)], yc[(1, 0)], yc[(1, 1)], 24, 24, 64)

    y3p = jnp.pad(y3, ((1, 1), (1, 1), (0, 0)))
    xc4 = _deconv_classes(y3p, 48, 48)
    oc = {}
    for i, (ry, rx) in enumerate([(0, 0), (0, 1), (1, 0), (1, 1)]):
        z = jnp.dot(xc4[(ry, rx)], w4_ref[i], preferred_element_type=f32)
        oc[(ry, rx)] = jnp.tanh(z + b4_ref[...]).reshape(48, 48, 3)
    y4 = _interleave(oc[(0, 0)], oc[(0, 1)], oc[(1, 0)], oc[(1, 1)], 48, 48, 3)

    out_ref[0] = y4.transpose(2, 0, 1)


def _deconv_w_classes(wt, kin, kout):
    mats = []
    for ry in range(2):
        for rx in range(2):
            taps = []
            for dy in range(2):
                for dx in range(2):
                    taps.append(wt[:, :, (3 - ry) - 2 * dy, (3 - rx) - 2 * dx])
            mats.append(jnp.concatenate(taps, axis=0))
    return jnp.stack(mats)


def kernel(identity_image, landmark_sequence, w1, b1, w2, b2, w3, b3, w4, b4):
    B, T, _ = landmark_sequence.shape
    BT = B * T

    w1m = w1.transpose(2, 3, 1, 0).reshape(64, 64)
    w2m = w2.transpose(2, 3, 1, 0).reshape(1024, 128)
    w3m = _deconv_w_classes(w3, 128, 64)
    w4m = _deconv_w_classes(w4, 64, 3)
    lmt = landmark_sequence.reshape(BT, 68, 2).transpose(0, 2, 1)

    grid = (BT,)
    out = pl.pallas_call(
        _frame_kernel,
        grid=grid,
        in_specs=[
            pl.BlockSpec((1, 3, HW, HW), lambda i: (i // T, 0, 0, 0)),
            pl.BlockSpec((1, 2, 68), lambda i: (i, 0, 0)),
            pl.BlockSpec((64, 64), lambda i: (0, 0)),
            pl.BlockSpec((1, 64), lambda i: (0, 0)),
            pl.BlockSpec((1024, 128), lambda i: (0, 0)),
            pl.BlockSpec((1, 128), lambda i: (0, 0)),
            pl.BlockSpec((4, 512, 64), lambda i: (0, 0, 0)),
            pl.BlockSpec((1, 64), lambda i: (0, 0)),
            pl.BlockSpec((4, 256, 3), lambda i: (0, 0, 0)),
            pl.BlockSpec((1, 3), lambda i: (0, 0)),
        ],
        out_specs=pl.BlockSpec((1, 3, HW, HW), lambda i: (i, 0, 0, 0)),
        out_shape=jax.ShapeDtypeStruct((BT, 3, HW, HW), jnp.float32),
        compiler_params=pltpu.CompilerParams(
            dimension_semantics=("parallel",),
        ),
    )(identity_image, lmt, w1m, b1.reshape(1, 64), w2m, b2.reshape(1, 128),
      w3m, b3.reshape(1, 64), w4m, b4.reshape(1, 3))
    return out.reshape(B, T, 3, HW, HW)


# speedup vs baseline: 1.2721x; 1.1725x over previous
import jax
import jax.numpy as jnp
from jax.experimental import pallas as pl
from jax.experimental.pallas import tpu as pltpu

HW = 96


def _im2col_s2(xp, oh, ow, k):
    H, W, C = xp.shape
    r = xp.reshape(H // 2, 2, W // 2, 2, C)
    planes = {(py, px): r[:, py, :, px, :]
              for py in (0, 1) for px in (0, 1)}
    cols = []
    for ky in range(k):
        for kx in range(k):
            p = planes[(ky % 2, kx % 2)]
            oy, ox = ky // 2, kx // 2
            cols.append(p[oy:oy + oh, ox:ox + ow, :].reshape(oh * ow, -1))
    return jnp.concatenate(cols, axis=1)


def _deconv_classes(xp, oh, ow):
    out = {}
    for ry in range(2):
        for rx in range(2):
            cols = []
            for dy in range(2):
                for dx in range(2):
                    cols.append(xp[ry + dy:ry + dy + oh, rx + dx:rx + dx + ow, :]
                                .reshape(oh * ow, -1))
            out[(ry, rx)] = jnp.concatenate(cols, axis=1)
    return out


def _interleave(c00, c01, c10, c11, h, w, ch):
    r0 = jnp.stack([c00, c01], axis=2)
    r1 = jnp.stack([c10, c11], axis=2)
    return jnp.stack([r0, r1], axis=1).reshape(2 * h, 2 * w, ch)


def _frame_kernel(ident_ref, lm_ref, w1_ref, b1_ref, w2_ref, b2_ref,
                  w3_ref, b3_ref, w4_ref, b4_ref, out_ref):
    f32 = jnp.float32

    lm = lm_ref[0] * (HW - 1)
    idx = lm.astype(jnp.int32)
    x, y = idx[0:1, :], idx[1:2, :]
    valid = (y >= 0) & (y < HW) & (x >= 0) & (x < HW)
    y_s = jnp.where(valid, y, -1)
    x_s = jnp.where(valid, x, -1)
    rows = jax.lax.broadcasted_iota(jnp.int32, (HW, 68), 0)
    rowm = (rows == y_s).astype(f32)
    colm = (rows == x_s).astype(f32)
    counts = jnp.dot(rowm, colm.T, preferred_element_type=f32)
    hm = (counts > 0).astype(f32)

    ident = ident_ref[0].transpose(1, 2, 0)
    x0 = jnp.concatenate([ident, hm[:, :, None]], axis=2)

    xp = jnp.pad(x0, ((1, 1), (1, 1), (0, 0)))
    x1 = _im2col_s2(xp, 48, 48, 4)
    y1 = jnp.maximum(jnp.dot(x1, w1_ref[...], preferred_element_type=f32)
                     + b1_ref[...], 0.0)
    y1 = y1.reshape(48, 48, 64)

    y1p = jnp.pad(y1, ((1, 1), (1, 1), (0, 0)))
    x2 = _im2col_s2(y1p, 24, 24, 4)
    y2 = jnp.maximum(jnp.dot(x2, w2_ref[...], preferred_element_type=f32)
                     + b2_ref[...], 0.0)
    y2 = y2.reshape(24, 24, 128)

    y2p = jnp.pad(y2, ((1, 1), (1, 1), (0, 0)))
    xc = _deconv_classes(y2p, 24, 24)
    yc = {}
    for i, (ry, rx) in enumerate([(0, 0), (0, 1), (1, 0), (1, 1)]):
        z = jnp.dot(xc[(ry, rx)], w3_ref[i], preferred_element_type=f32)
        yc[(ry, rx)] = jnp.maximum(z + b3_ref[...], 0.0).reshape(24, 24, 64)
    y3 = _interleave(yc[(0, 0)], yc[(0, 1)], yc[(1, 0)], yc[(1, 1)], 24, 24, 64)

    y3p = jnp.pad(y3, ((1, 1), (1, 1), (0, 0)))
    xc4 = _deconv_classes(y3p, 48, 48)
    x4 = jnp.concatenate([xc4[(0, 0)], xc4[(0, 1)], xc4[(1, 0)], xc4[(1, 1)]],
                         axis=1)
    z = jnp.dot(x4, w4_ref[...], preferred_element_type=f32) + b4_ref[...]
    out_ref[0] = jnp.tanh(z).reshape(48, 48, 12)


def _deconv_w_classes(wt, kin, kout):
    mats = []
    for ry in range(2):
        for rx in range(2):
            taps = []
            for dy in range(2):
                for dx in range(2):
                    taps.append(wt[:, :, (3 - ry) - 2 * dy, (3 - rx) - 2 * dx])
            mats.append(jnp.concatenate(taps, axis=0))
    return jnp.stack(mats)


def kernel(identity_image, landmark_sequence, w1, b1, w2, b2, w3, b3, w4, b4):
    B, T, _ = landmark_sequence.shape
    BT = B * T

    w1m = w1.transpose(2, 3, 1, 0).reshape(64, 64)
    w2m = w2.transpose(2, 3, 1, 0).reshape(1024, 128)
    w3m = _deconv_w_classes(w3, 128, 64)
    w4c = _deconv_w_classes(w4, 64, 3)
    w4m = jax.scipy.linalg.block_diag(*[w4c[i] for i in range(4)])
    b4m = jnp.tile(b4, 4).reshape(1, 12)
    lmt = landmark_sequence.reshape(BT, 68, 2).transpose(0, 2, 1)

    grid = (BT,)
    out = pl.pallas_call(
        _frame_kernel,
        grid=grid,
        in_specs=[
            pl.BlockSpec((1, 3, HW, HW), lambda i: (i // T, 0, 0, 0)),
            pl.BlockSpec((1, 2, 68), lambda i: (i, 0, 0)),
            pl.BlockSpec((64, 64), lambda i: (0, 0)),
            pl.BlockSpec((1, 64), lambda i: (0, 0)),
            pl.BlockSpec((1024, 128), lambda i: (0, 0)),
            pl.BlockSpec((1, 128), lambda i: (0, 0)),
            pl.BlockSpec((4, 512, 64), lambda i: (0, 0, 0)),
            pl.BlockSpec((1, 64), lambda i: (0, 0)),
            pl.BlockSpec((1024, 12), lambda i: (0, 0)),
            pl.BlockSpec((1, 12), lambda i: (0, 0)),
        ],
        out_specs=pl.BlockSpec((1, 48, 48, 12), lambda i: (i, 0, 0, 0)),
        out_shape=jax.ShapeDtypeStruct((BT, 48, 48, 12), jnp.float32),
        compiler_params=pltpu.CompilerParams(
            dimension_semantics=("parallel",),
        ),
    )(identity_image, lmt, w1m, b1.reshape(1, 64), w2m, b2.reshape(1, 128),
      w3m, b3.reshape(1, 64), w4m, b4m)
    out = out.reshape(BT, 48, 48, 2, 2, 3).transpose(0, 5, 1, 3, 2, 4)
    return out.reshape(B, T, 3, HW, HW)
